# Initial kernel scaffold; baseline (speedup 1.0000x reference)
import jax
import jax.numpy as jnp
from jax.experimental import pallas as pl


def kernel(x, edge_index, batch, W0_0, b0_0, W0_1, b0_1, W1_0, b1_0, W1_1, b1_1, W2_0, b2_0, W2_1, b2_1, Wc1, bc1, Wc2, bc2):
    raise NotImplementedError("write your pallas kernel here")


# speedup vs baseline: 9.1299x; 9.1299x over previous
import functools

import jax
import jax.numpy as jnp
from jax import lax
from jax.experimental import pallas as pl
from jax.experimental.pallas import tpu as pltpu
from jax.experimental.pallas import tpu_sc as plsc

_N, _E, _D, _H, _O, _G = 10000, 320000, 128, 32, 2, 64
_BM = 400
_GRID = _N // _BM
_NW = 32
_CHUNK = 128
_KCH = 80
_EPT = _CHUNK * _KCH
_EPAD = _NW * _EPT
_NPAD = 10112
_RPT = _NPAD // 16
_NBUF = 2


def _proj_body(x_ref, w_ref, o_ref):
    o_ref[...] = jnp.dot(x_ref[...], w_ref[...],
                         preferred_element_type=jnp.float32)


def _project(x, w0):
    return pl.pallas_call(
        _proj_body,
        grid=(_GRID,),
        in_specs=[
            pl.BlockSpec((_BM, _D), lambda i: (i, 0)),
            pl.BlockSpec((_D, _H), lambda i: (0, 0)),
        ],
        out_specs=pl.BlockSpec((_BM, _H), lambda i: (i, 0)),
        out_shape=jax.ShapeDtypeStruct((_N, _H), jnp.float32),
    )(x, w0)


def _mid_body(y_ref, p0_ref, p1_ref, ba_ref, wb_ref, bb_ref, wn_ref, o_ref):
    m = jnp.maximum(y_ref[...] + p0_ref[...] + p1_ref[...] + ba_ref[...], 0.0)
    h = jnp.maximum(
        jnp.dot(m, wb_ref[...], preferred_element_type=jnp.float32)
        + bb_ref[...], 0.0)
    o_ref[...] = jnp.dot(h, wn_ref[...], preferred_element_type=jnp.float32)


def _mid(y, p0, p1, ba, wb, bb, wn):
    return pl.pallas_call(
        _mid_body,
        grid=(_GRID,),
        in_specs=[
            pl.BlockSpec((_BM, _H), lambda i: (i, 0)),
            pl.BlockSpec((_BM, _H), lambda i: (i, 0)),
            pl.BlockSpec((_BM, _H), lambda i: (i, 0)),
            pl.BlockSpec((1, _H), lambda i: (0, 0)),
            pl.BlockSpec((_H, _H), lambda i: (0, 0)),
            pl.BlockSpec((1, _H), lambda i: (0, 0)),
            pl.BlockSpec((_H, _H), lambda i: (0, 0)),
        ],
        out_specs=pl.BlockSpec((_BM, _H), lambda i: (i, 0)),
        out_shape=jax.ShapeDtypeStruct((_N, _H), jnp.float32),
    )(y, p0, p1, ba.reshape(1, _H), wb, bb.reshape(1, _H), wn)


def _final_body(y_ref, p0_ref, p1_ref, ba_ref, wb_ref, bb_ref, batch_ref,
                wc1_ref, bc1_ref, wc2_ref, bc2_ref, o_ref, sums_scr, cnts_scr):
    i = pl.program_id(0)

    @pl.when(i == 0)
    def _():
        sums_scr[...] = jnp.zeros_like(sums_scr)
        cnts_scr[...] = jnp.zeros_like(cnts_scr)

    m = jnp.maximum(y_ref[...] + p0_ref[...] + p1_ref[...] + ba_ref[...], 0.0)
    h = jnp.maximum(
        jnp.dot(m, wb_ref[...], preferred_element_type=jnp.float32)
        + bb_ref[...], 0.0)
    b = batch_ref[0, 0, :]
    oht = (b[None, :] == lax.broadcasted_iota(jnp.int32, (_G, _BM), 0)
           ).astype(jnp.float32)
    sums_scr[...] += jnp.dot(oht, h, preferred_element_type=jnp.float32)
    cnts_scr[...] += jnp.sum(oht, axis=1)[:, None]

    @pl.when(i == _GRID - 1)
    def _():
        pooled = sums_scr[...] / jnp.maximum(cnts_scr[...], 1.0)
        z1 = jnp.maximum(
            jnp.dot(pooled, wc1_ref[...], preferred_element_type=jnp.float32)
            + bc1_ref[...], 0.0)
        z = jnp.dot(z1, wc2_ref[...],
                    preferred_element_type=jnp.float32) + bc2_ref[...]
        mx = jnp.max(z, axis=1, keepdims=True)
        e = jnp.exp(z - mx)
        o_ref[...] = z - mx - jnp.log(jnp.sum(e, axis=1, keepdims=True))


def _final(y, p0, p1, ba, wb, bb, batch_r, wc1, bc1, wc2, bc2):
    return pl.pallas_call(
        _final_body,
        grid=(_GRID,),
        in_specs=[
            pl.BlockSpec((_BM, _H), lambda i: (i, 0)),
            pl.BlockSpec((_BM, _H), lambda i: (i, 0)),
            pl.BlockSpec((_BM, _H), lambda i: (i, 0)),
            pl.BlockSpec((1, _H), lambda i: (0, 0)),
            pl.BlockSpec((_H, _H), lambda i: (0, 0)),
            pl.BlockSpec((1, _H), lambda i: (0, 0)),
            pl.BlockSpec((1, 1, _BM), lambda i: (i, 0, 0)),
            pl.BlockSpec((_H, _H), lambda i: (0, 0)),
            pl.BlockSpec((1, _H), lambda i: (0, 0)),
            pl.BlockSpec((_H, _O), lambda i: (0, 0)),
            pl.BlockSpec((1, _O), lambda i: (0, 0)),
        ],
        out_specs=pl.BlockSpec((_G, _O), lambda i: (0, 0)),
        out_shape=jax.ShapeDtypeStruct((_G, _O), jnp.float32),
        scratch_shapes=[
            pltpu.VMEM((_G, _H), jnp.float32),
            pltpu.VMEM((_G, 1), jnp.float32),
        ],
    )(y, p0, p1, ba.reshape(1, _H), wb, bb.reshape(1, _H), batch_r,
      wc1, bc1.reshape(1, _H), wc2, bc2.reshape(1, _O))


_sc_mesh = plsc.VectorSubcoreMesh(core_axis_name="c", subcore_axis_name="s")


@functools.partial(
    pl.kernel,
    mesh=_sc_mesh,
    compiler_params=pltpu.CompilerParams(use_tc_tiling_on_sc=False),
    out_type=(jax.ShapeDtypeStruct((_NPAD, _H), jnp.float32),
              jax.ShapeDtypeStruct((_NPAD, _H), jnp.float32)),
    scratch_types=[
        pltpu.VMEM((_KCH, _CHUNK), jnp.int32),
        pltpu.VMEM((_KCH, _CHUNK), jnp.int32),
        pltpu.VMEM((_NBUF, _CHUNK, _H), jnp.float32),
        pltpu.VMEM_SHARED((_NPAD, _H), jnp.float32),
        pltpu.SemaphoreType.DMA,
        pltpu.SemaphoreType.DMA,
    ],
)
def _sc_agg(y_hbm, zeros_hbm, src_hbm, dst_hbm, out0_hbm, out1_hbm,
            sidx, didx, rows, acc, sem0, sem1):
    cid = lax.axis_index("c")
    sid = lax.axis_index("s")
    w = sid * 2 + cid
    r0 = sid * _RPT

    pltpu.sync_copy(zeros_hbm.at[pl.ds(r0, _RPT)], acc.at[pl.ds(r0, _RPT)])
    pltpu.sync_copy(src_hbm.at[pl.ds(w * _KCH, _KCH)], sidx)
    pltpu.sync_copy(dst_hbm.at[pl.ds(w * _KCH, _KCH)], didx)
    plsc.subcore_barrier()

    sems = (sem0, sem1)
    for b in range(_NBUF):
        pltpu.async_copy(y_hbm.at[sidx.at[b]], rows.at[b], sems[b])

    def body(it, carry):
        base = it * _NBUF
        for b in range(_NBUF):
            j = base + b
            pltpu.make_async_copy(y_hbm.at[sidx.at[j]], rows.at[b],
                                  sems[b]).wait()
            pltpu.sync_copy(rows.at[b], acc.at[didx.at[j]], add=True)
            nj = j + _NBUF

            @pl.when(nj < _KCH)
            def _():
                pltpu.async_copy(y_hbm.at[sidx.at[nj]], rows.at[b], sems[b])
        return carry

    lax.fori_loop(0, _KCH // _NBUF, body, 0)
    plsc.subcore_barrier()

    @pl.when(cid == 0)
    def _():
        pltpu.sync_copy(acc.at[pl.ds(r0, _RPT)], out0_hbm.at[pl.ds(r0, _RPT)])

    @pl.when(cid == 1)
    def _():
        pltpu.sync_copy(acc.at[pl.ds(r0, _RPT)], out1_hbm.at[pl.ds(r0, _RPT)])


def kernel(x, edge_index, batch, W0_0, b0_0, W0_1, b0_1, W1_0, b1_0,
           W1_1, b1_1, W2_0, b2_0, W2_1, b2_1, Wc1, bc1, Wc2, bc2):
    src = edge_index[0]
    dst = edge_index[1]
    pad = _EPAD - _E
    srcp = jnp.concatenate(
        [src, jnp.zeros((pad,), jnp.int32)]).reshape(_NW * _KCH, _CHUNK)
    dstp = jnp.concatenate(
        [dst, jnp.full((pad,), _N, jnp.int32)]).reshape(_NW * _KCH, _CHUNK)
    zeros = jnp.zeros((_NPAD, _H), jnp.float32)
    batch_r = batch.reshape(_GRID, 1, _BM)

    y0 = _project(x, W0_0)
    p0a, p0b = _sc_agg(y0, zeros, srcp, dstp)
    y1 = _mid(y0, p0a, p0b, b0_0, W0_1, b0_1, W1_0)
    p1a, p1b = _sc_agg(y1, zeros, srcp, dstp)
    y2 = _mid(y1, p1a, p1b, b1_0, W1_1, b1_1, W2_0)
    p2a, p2b = _sc_agg(y2, zeros, srcp, dstp)
    return _final(y2, p2a, p2b, b2_0, W2_1, b2_1, batch_r, Wc1, bc1, Wc2, bc2)


# speedup vs baseline: 9.2978x; 1.0184x over previous
import functools

import jax
import jax.numpy as jnp
from jax import lax
from jax.experimental import pallas as pl
from jax.experimental.pallas import tpu as pltpu
from jax.experimental.pallas import tpu_sc as plsc

_N, _E, _D, _H, _O, _G = 10000, 320000, 128, 32, 2, 64
_BM = 400
_GRID = _N // _BM
_NW = 32
_CHUNK = 128
_KCH = 80
_EPT = _CHUNK * _KCH
_EPAD = _NW * _EPT
_NPAD = 10112
_RPT = _NPAD // 16
_NBUF = 8
_DEPTH = 4


def _proj_body(x_ref, w_ref, o_ref):
    o_ref[...] = jnp.dot(x_ref[...], w_ref[...],
                         preferred_element_type=jnp.float32)


def _project(x, w0):
    return pl.pallas_call(
        _proj_body,
        grid=(_GRID,),
        in_specs=[
            pl.BlockSpec((_BM, _D), lambda i: (i, 0)),
            pl.BlockSpec((_D, _H), lambda i: (0, 0)),
        ],
        out_specs=pl.BlockSpec((_BM, _H), lambda i: (i, 0)),
        out_shape=jax.ShapeDtypeStruct((_N, _H), jnp.float32),
    )(x, w0)


def _mid_body(y_ref, p0_ref, p1_ref, ba_ref, wb_ref, bb_ref, wn_ref, o_ref):
    m = jnp.maximum(y_ref[...] + p0_ref[...] + p1_ref[...] + ba_ref[...], 0.0)
    h = jnp.maximum(
        jnp.dot(m, wb_ref[...], preferred_element_type=jnp.float32)
        + bb_ref[...], 0.0)
    o_ref[...] = jnp.dot(h, wn_ref[...], preferred_element_type=jnp.float32)


def _mid(y, p0, p1, ba, wb, bb, wn):
    return pl.pallas_call(
        _mid_body,
        grid=(_GRID,),
        in_specs=[
            pl.BlockSpec((_BM, _H), lambda i: (i, 0)),
            pl.BlockSpec((_BM, _H), lambda i: (i, 0)),
            pl.BlockSpec((_BM, _H), lambda i: (i, 0)),
            pl.BlockSpec((1, _H), lambda i: (0, 0)),
            pl.BlockSpec((_H, _H), lambda i: (0, 0)),
            pl.BlockSpec((1, _H), lambda i: (0, 0)),
            pl.BlockSpec((_H, _H), lambda i: (0, 0)),
        ],
        out_specs=pl.BlockSpec((_BM, _H), lambda i: (i, 0)),
        out_shape=jax.ShapeDtypeStruct((_N, _H), jnp.float32),
    )(y, p0, p1, ba.reshape(1, _H), wb, bb.reshape(1, _H), wn)


def _final_body(y_ref, p0_ref, p1_ref, ba_ref, wb_ref, bb_ref, batch_ref,
                wc1_ref, bc1_ref, wc2_ref, bc2_ref, o_ref, sums_scr, cnts_scr):
    i = pl.program_id(0)

    @pl.when(i == 0)
    def _():
        sums_scr[...] = jnp.zeros_like(sums_scr)
        cnts_scr[...] = jnp.zeros_like(cnts_scr)

    m = jnp.maximum(y_ref[...] + p0_ref[...] + p1_ref[...] + ba_ref[...], 0.0)
    h = jnp.maximum(
        jnp.dot(m, wb_ref[...], preferred_element_type=jnp.float32)
        + bb_ref[...], 0.0)
    b = batch_ref[0, 0, :]
    oht = (b[None, :] == lax.broadcasted_iota(jnp.int32, (_G, _BM), 0)
           ).astype(jnp.float32)
    sums_scr[...] += jnp.dot(oht, h, preferred_element_type=jnp.float32)
    cnts_scr[...] += jnp.sum(oht, axis=1)[:, None]

    @pl.when(i == _GRID - 1)
    def _():
        pooled = sums_scr[...] / jnp.maximum(cnts_scr[...], 1.0)
        z1 = jnp.maximum(
            jnp.dot(pooled, wc1_ref[...], preferred_element_type=jnp.float32)
            + bc1_ref[...], 0.0)
        z = jnp.dot(z1, wc2_ref[...],
                    preferred_element_type=jnp.float32) + bc2_ref[...]
        mx = jnp.max(z, axis=1, keepdims=True)
        e = jnp.exp(z - mx)
        o_ref[...] = z - mx - jnp.log(jnp.sum(e, axis=1, keepdims=True))


def _final(y, p0, p1, ba, wb, bb, batch_r, wc1, bc1, wc2, bc2):
    return pl.pallas_call(
        _final_body,
        grid=(_GRID,),
        in_specs=[
            pl.BlockSpec((_BM, _H), lambda i: (i, 0)),
            pl.BlockSpec((_BM, _H), lambda i: (i, 0)),
            pl.BlockSpec((_BM, _H), lambda i: (i, 0)),
            pl.BlockSpec((1, _H), lambda i: (0, 0)),
            pl.BlockSpec((_H, _H), lambda i: (0, 0)),
            pl.BlockSpec((1, _H), lambda i: (0, 0)),
            pl.BlockSpec((1, 1, _BM), lambda i: (i, 0, 0)),
            pl.BlockSpec((_H, _H), lambda i: (0, 0)),
            pl.BlockSpec((1, _H), lambda i: (0, 0)),
            pl.BlockSpec((_H, _O), lambda i: (0, 0)),
            pl.BlockSpec((1, _O), lambda i: (0, 0)),
        ],
        out_specs=pl.BlockSpec((_G, _O), lambda i: (0, 0)),
        out_shape=jax.ShapeDtypeStruct((_G, _O), jnp.float32),
        scratch_shapes=[
            pltpu.VMEM((_G, _H), jnp.float32),
            pltpu.VMEM((_G, 1), jnp.float32),
        ],
    )(y, p0, p1, ba.reshape(1, _H), wb, bb.reshape(1, _H), batch_r,
      wc1, bc1.reshape(1, _H), wc2, bc2.reshape(1, _O))


_sc_mesh = plsc.VectorSubcoreMesh(core_axis_name="c", subcore_axis_name="s")


@functools.partial(
    pl.kernel,
    mesh=_sc_mesh,
    compiler_params=pltpu.CompilerParams(use_tc_tiling_on_sc=False),
    out_type=(jax.ShapeDtypeStruct((_NPAD, _H), jnp.float32),
              jax.ShapeDtypeStruct((_NPAD, _H), jnp.float32)),
    scratch_types=[
        pltpu.VMEM((_KCH, _CHUNK), jnp.int32),
        pltpu.VMEM((_KCH, _CHUNK), jnp.int32),
        pltpu.VMEM((_NBUF, _CHUNK, _H), jnp.float32),
        pltpu.VMEM_SHARED((_NPAD, _H), jnp.float32),
        pltpu.SemaphoreType.DMA,
        pltpu.SemaphoreType.DMA,
    ],
)
def _sc_agg(y_hbm, zeros_hbm, src_hbm, dst_hbm, out0_hbm, out1_hbm,
            sidx, didx, rows, acc, gsem, ssem):
    cid = lax.axis_index("c")
    sid = lax.axis_index("s")
    w = sid * 2 + cid
    r0 = sid * _RPT

    pltpu.sync_copy(zeros_hbm.at[pl.ds(r0, _RPT)], acc.at[pl.ds(r0, _RPT)])
    pltpu.sync_copy(src_hbm.at[pl.ds(w * _KCH, _KCH)], sidx)
    pltpu.sync_copy(dst_hbm.at[pl.ds(w * _KCH, _KCH)], didx)
    plsc.subcore_barrier()

    for b in range(_DEPTH):
        pltpu.async_copy(y_hbm.at[sidx.at[b]], rows.at[b], gsem)

    def body(it, carry):
        base = it * _NBUF
        for b in range(_NBUF):
            j = base + b
            pltpu.make_async_copy(y_hbm.at[sidx.at[j]], rows.at[b],
                                  gsem).wait()
            pltpu.async_copy(rows.at[b], acc.at[didx.at[j]], ssem, add=True)

            @pl.when(j >= _DEPTH)
            def _():
                pltpu.make_async_copy(rows.at[b], acc.at[didx.at[j]],
                                      ssem).wait()

            nj = j + _DEPTH
            nb = (b + _DEPTH) % _NBUF

            @pl.when(nj < _KCH)
            def _():
                pltpu.async_copy(y_hbm.at[sidx.at[nj]], rows.at[nb], gsem)
        return carry

    lax.fori_loop(0, _KCH // _NBUF, body, 0)
    for _ in range(_DEPTH):
        pltpu.make_async_copy(rows.at[0], acc.at[didx.at[0]], ssem).wait()
    plsc.subcore_barrier()

    @pl.when(cid == 0)
    def _():
        pltpu.sync_copy(acc.at[pl.ds(r0, _RPT)], out0_hbm.at[pl.ds(r0, _RPT)])

    @pl.when(cid == 1)
    def _():
        pltpu.sync_copy(acc.at[pl.ds(r0, _RPT)], out1_hbm.at[pl.ds(r0, _RPT)])


def kernel(x, edge_index, batch, W0_0, b0_0, W0_1, b0_1, W1_0, b1_0,
           W1_1, b1_1, W2_0, b2_0, W2_1, b2_1, Wc1, bc1, Wc2, bc2):
    src = edge_index[0]
    dst = edge_index[1]
    pad = _EPAD - _E
    srcp = jnp.concatenate(
        [src, jnp.zeros((pad,), jnp.int32)]).reshape(_NW * _KCH, _CHUNK)
    trash = _N + (jnp.arange(pad, dtype=jnp.int32) % (_NPAD - _N))
    dstp = jnp.concatenate([dst, trash]).reshape(_NW * _KCH, _CHUNK)
    zeros = jnp.zeros((_NPAD, _H), jnp.float32)
    batch_r = batch.reshape(_GRID, 1, _BM)

    y0 = _project(x, W0_0)
    p0a, p0b = _sc_agg(y0, zeros, srcp, dstp)
    y1 = _mid(y0, p0a, p0b, b0_0, W0_1, b0_1, W1_0)
    p1a, p1b = _sc_agg(y1, zeros, srcp, dstp)
    y2 = _mid(y1, p1a, p1b, b1_0, W1_1, b1_1, W2_0)
    p2a, p2b = _sc_agg(y2, zeros, srcp, dstp)
    return _final(y2, p2a, p2b, b2_0, W2_1, b2_1, batch_r, Wc1, bc1, Wc2, bc2)


# speedup vs baseline: 16.4208x; 1.7661x over previous
import functools

import jax
import jax.numpy as jnp
from jax import lax
from jax.experimental import pallas as pl
from jax.experimental.pallas import tpu as pltpu
from jax.experimental.pallas import tpu_sc as plsc

_N, _E, _D, _H, _O, _G = 10000, 320000, 128, 32, 2, 64
_BM = 400
_GRID = _N // _BM
_NW = 32
_CHUNK = 128
_KCH = 80
_EPT = _CHUNK * _KCH
_EPAD = _NW * _EPT
_NPAD = 10112
_RPT = _NPAD // 16
_NBUF = 8
_DEPTH = 4


def _proj_body(x_ref, w_ref, o_ref):
    o_ref[...] = jnp.dot(x_ref[...], w_ref[...],
                         preferred_element_type=jnp.float32)


def _project(x, w0):
    return pl.pallas_call(
        _proj_body,
        grid=(_GRID,),
        in_specs=[
            pl.BlockSpec((_BM, _D), lambda i: (i, 0)),
            pl.BlockSpec((_D, _H), lambda i: (0, 0)),
        ],
        out_specs=pl.BlockSpec((_BM, _H), lambda i: (i, 0)),
        out_shape=jax.ShapeDtypeStruct((_N, _H), jnp.float32),
    )(x, w0)


def _mid_body(y_ref, p0_ref, p1_ref, ba_ref, wb_ref, bb_ref, wn_ref, o_ref):
    m = jnp.maximum(y_ref[...] + p0_ref[...] + p1_ref[...] + ba_ref[...], 0.0)
    h = jnp.maximum(
        jnp.dot(m, wb_ref[...], preferred_element_type=jnp.float32)
        + bb_ref[...], 0.0)
    o_ref[...] = jnp.dot(h, wn_ref[...], preferred_element_type=jnp.float32)


def _mid(y, p0, p1, ba, wb, bb, wn):
    return pl.pallas_call(
        _mid_body,
        grid=(_GRID,),
        in_specs=[
            pl.BlockSpec((_BM, _H), lambda i: (i, 0)),
            pl.BlockSpec((_BM, _H), lambda i: (i, 0)),
            pl.BlockSpec((_BM, _H), lambda i: (i, 0)),
            pl.BlockSpec((1, _H), lambda i: (0, 0)),
            pl.BlockSpec((_H, _H), lambda i: (0, 0)),
            pl.BlockSpec((1, _H), lambda i: (0, 0)),
            pl.BlockSpec((_H, _H), lambda i: (0, 0)),
        ],
        out_specs=pl.BlockSpec((_BM, _H), lambda i: (i, 0)),
        out_shape=jax.ShapeDtypeStruct((_N, _H), jnp.float32),
    )(y, p0, p1, ba.reshape(1, _H), wb, bb.reshape(1, _H), wn)


def _final_body(y_ref, p0_ref, p1_ref, ba_ref, wb_ref, bb_ref, batch_ref,
                wc1_ref, bc1_ref, wc2_ref, bc2_ref, o_ref, sums_scr, cnts_scr):
    i = pl.program_id(0)

    @pl.when(i == 0)
    def _():
        sums_scr[...] = jnp.zeros_like(sums_scr)
        cnts_scr[...] = jnp.zeros_like(cnts_scr)

    m = jnp.maximum(y_ref[...] + p0_ref[...] + p1_ref[...] + ba_ref[...], 0.0)
    h = jnp.maximum(
        jnp.dot(m, wb_ref[...], preferred_element_type=jnp.float32)
        + bb_ref[...], 0.0)
    b = batch_ref[0, 0, :]
    oht = (b[None, :] == lax.broadcasted_iota(jnp.int32, (_G, _BM), 0)
           ).astype(jnp.float32)
    sums_scr[...] += jnp.dot(oht, h, preferred_element_type=jnp.float32)
    cnts_scr[...] += jnp.sum(oht, axis=1)[:, None]

    @pl.when(i == _GRID - 1)
    def _():
        pooled = sums_scr[...] / jnp.maximum(cnts_scr[...], 1.0)
        z1 = jnp.maximum(
            jnp.dot(pooled, wc1_ref[...], preferred_element_type=jnp.float32)
            + bc1_ref[...], 0.0)
        z = jnp.dot(z1, wc2_ref[...],
                    preferred_element_type=jnp.float32) + bc2_ref[...]
        mx = jnp.max(z, axis=1, keepdims=True)
        e = jnp.exp(z - mx)
        o_ref[...] = z - mx - jnp.log(jnp.sum(e, axis=1, keepdims=True))


def _final(y, p0, p1, ba, wb, bb, batch_r, wc1, bc1, wc2, bc2):
    return pl.pallas_call(
        _final_body,
        grid=(_GRID,),
        in_specs=[
            pl.BlockSpec((_BM, _H), lambda i: (i, 0)),
            pl.BlockSpec((_BM, _H), lambda i: (i, 0)),
            pl.BlockSpec((_BM, _H), lambda i: (i, 0)),
            pl.BlockSpec((1, _H), lambda i: (0, 0)),
            pl.BlockSpec((_H, _H), lambda i: (0, 0)),
            pl.BlockSpec((1, _H), lambda i: (0, 0)),
            pl.BlockSpec((1, 1, _BM), lambda i: (i, 0, 0)),
            pl.BlockSpec((_H, _H), lambda i: (0, 0)),
            pl.BlockSpec((1, _H), lambda i: (0, 0)),
            pl.BlockSpec((_H, _O), lambda i: (0, 0)),
            pl.BlockSpec((1, _O), lambda i: (0, 0)),
        ],
        out_specs=pl.BlockSpec((_G, _O), lambda i: (0, 0)),
        out_shape=jax.ShapeDtypeStruct((_G, _O), jnp.float32),
        scratch_shapes=[
            pltpu.VMEM((_G, _H), jnp.float32),
            pltpu.VMEM((_G, 1), jnp.float32),
        ],
    )(y, p0, p1, ba.reshape(1, _H), wb, bb.reshape(1, _H), batch_r,
      wc1, bc1.reshape(1, _H), wc2, bc2.reshape(1, _O))


_sc_mesh = plsc.VectorSubcoreMesh(core_axis_name="c", subcore_axis_name="s")


@functools.partial(
    pl.kernel,
    mesh=_sc_mesh,
    compiler_params=pltpu.CompilerParams(use_tc_tiling_on_sc=False),
    out_type=(jax.ShapeDtypeStruct((_NPAD, _H), jnp.float32),
              jax.ShapeDtypeStruct((_NPAD, _H), jnp.float32)),
    scratch_types=[
        pltpu.VMEM((_KCH, _CHUNK), jnp.int32),
        pltpu.VMEM((_KCH, _CHUNK), jnp.int32),
        pltpu.VMEM((_NBUF, _CHUNK, _H), jnp.float32),
        pltpu.VMEM_SHARED((_NPAD, _H), jnp.float32),
        pltpu.VMEM_SHARED((_N, _H), jnp.float32),
        pltpu.SemaphoreType.DMA,
        pltpu.SemaphoreType.DMA,
    ],
)
def _sc_agg(y_hbm, zeros_hbm, src_hbm, dst_hbm, out0_hbm, out1_hbm,
            sidx, didx, rows, acc, y_sh, gsem, ssem):
    cid = lax.axis_index("c")
    sid = lax.axis_index("s")
    w = sid * 2 + cid
    r0 = sid * _RPT
    ry = _N // 16

    pltpu.sync_copy(zeros_hbm.at[pl.ds(r0, _RPT)], acc.at[pl.ds(r0, _RPT)])
    pltpu.sync_copy(y_hbm.at[pl.ds(sid * ry, ry)], y_sh.at[pl.ds(sid * ry, ry)])
    pltpu.sync_copy(src_hbm.at[pl.ds(w * _KCH, _KCH)], sidx)
    pltpu.sync_copy(dst_hbm.at[pl.ds(w * _KCH, _KCH)], didx)
    plsc.subcore_barrier()

    for b in range(_DEPTH):
        pltpu.async_copy(y_sh.at[sidx.at[b]], rows.at[b], gsem)

    def body(it, carry):
        base = it * _NBUF
        for b in range(_NBUF):
            j = base + b
            pltpu.make_async_copy(y_sh.at[sidx.at[j]], rows.at[b],
                                  gsem).wait()
            pltpu.async_copy(rows.at[b], acc.at[didx.at[j]], ssem, add=True)

            @pl.when(j >= _DEPTH)
            def _():
                pltpu.make_async_copy(rows.at[b], acc.at[didx.at[j]],
                                      ssem).wait()

            nj = j + _DEPTH
            nb = (b + _DEPTH) % _NBUF

            @pl.when(nj < _KCH)
            def _():
                pltpu.async_copy(y_sh.at[sidx.at[nj]], rows.at[nb], gsem)
        return carry

    lax.fori_loop(0, _KCH // _NBUF, body, 0)
    for _ in range(_DEPTH):
        pltpu.make_async_copy(rows.at[0], acc.at[didx.at[0]], ssem).wait()
    plsc.subcore_barrier()

    @pl.when(cid == 0)
    def _():
        pltpu.sync_copy(acc.at[pl.ds(r0, _RPT)], out0_hbm.at[pl.ds(r0, _RPT)])

    @pl.when(cid == 1)
    def _():
        pltpu.sync_copy(acc.at[pl.ds(r0, _RPT)], out1_hbm.at[pl.ds(r0, _RPT)])


def kernel(x, edge_index, batch, W0_0, b0_0, W0_1, b0_1, W1_0, b1_0,
           W1_1, b1_1, W2_0, b2_0, W2_1, b2_1, Wc1, bc1, Wc2, bc2):
    src = edge_index[0]
    dst = edge_index[1]
    pad = _EPAD - _E
    srcp = jnp.concatenate(
        [src, jnp.zeros((pad,), jnp.int32)]).reshape(_NW * _KCH, _CHUNK)
    trash = _N + (jnp.arange(pad, dtype=jnp.int32) % (_NPAD - _N))
    dstp = jnp.concatenate([dst, trash]).reshape(_NW * _KCH, _CHUNK)
    zeros = jnp.zeros((_NPAD, _H), jnp.float32)
    batch_r = batch.reshape(_GRID, 1, _BM)

    y0 = _project(x, W0_0)
    p0a, p0b = _sc_agg(y0, zeros, srcp, dstp)
    y1 = _mid(y0, p0a, p0b, b0_0, W0_1, b0_1, W1_0)
    p1a, p1b = _sc_agg(y1, zeros, srcp, dstp)
    y2 = _mid(y1, p1a, p1b, b1_0, W1_1, b1_1, W2_0)
    p2a, p2b = _sc_agg(y2, zeros, srcp, dstp)
    return _final(y2, p2a, p2b, b2_0, W2_1, b2_1, batch_r, Wc1, bc1, Wc2, bc2)


# speedup vs baseline: 19.5424x; 1.1901x over previous
import functools

import jax
import jax.numpy as jnp
from jax import lax
from jax.experimental import pallas as pl
from jax.experimental.pallas import tpu as pltpu
from jax.experimental.pallas import tpu_sc as plsc

_N, _E, _D, _H, _O, _G = 10000, 320000, 128, 32, 2, 64
_BM = 2000
_GRID = _N // _BM
_NW = 32
_CHUNK = 128
_KCH = 80
_EPT = _CHUNK * _KCH
_EPAD = _NW * _EPT
_NPAD = 10112
_RPT = _NPAD // 16
_NBUF = 8
_DEPTH = 4


def _proj_body(x_ref, w_ref, o_ref):
    o_ref[...] = jnp.dot(x_ref[...], w_ref[...],
                         preferred_element_type=jnp.float32)


def _project(x, w0):
    return pl.pallas_call(
        _proj_body,
        grid=(_GRID,),
        in_specs=[
            pl.BlockSpec((_BM, _D), lambda i: (i, 0)),
            pl.BlockSpec((_D, _H), lambda i: (0, 0)),
        ],
        out_specs=pl.BlockSpec((_BM, _H), lambda i: (i, 0)),
        out_shape=jax.ShapeDtypeStruct((_N, _H), jnp.float32),
    )(x, w0)


def _mid_body(y_ref, p0_ref, p1_ref, ba_ref, wb_ref, bb_ref, wn_ref, o_ref):
    m = jnp.maximum(y_ref[...] + p0_ref[...] + p1_ref[...] + ba_ref[...], 0.0)
    h = jnp.maximum(
        jnp.dot(m, wb_ref[...], preferred_element_type=jnp.float32)
        + bb_ref[...], 0.0)
    o_ref[...] = jnp.dot(h, wn_ref[...], preferred_element_type=jnp.float32)


def _mid(y, p0, p1, ba, wb, bb, wn):
    return pl.pallas_call(
        _mid_body,
        grid=(_GRID,),
        in_specs=[
            pl.BlockSpec((_BM, _H), lambda i: (i, 0)),
            pl.BlockSpec((_BM, _H), lambda i: (i, 0)),
            pl.BlockSpec((_BM, _H), lambda i: (i, 0)),
            pl.BlockSpec((1, _H), lambda i: (0, 0)),
            pl.BlockSpec((_H, _H), lambda i: (0, 0)),
            pl.BlockSpec((1, _H), lambda i: (0, 0)),
            pl.BlockSpec((_H, _H), lambda i: (0, 0)),
        ],
        out_specs=pl.BlockSpec((_BM, _H), lambda i: (i, 0)),
        out_shape=jax.ShapeDtypeStruct((_N, _H), jnp.float32),
    )(y, p0, p1, ba.reshape(1, _H), wb, bb.reshape(1, _H), wn)


def _final_body(y_ref, p0_ref, p1_ref, ba_ref, wb_ref, bb_ref, batch_ref,
                wc1_ref, bc1_ref, wc2_ref, bc2_ref, o_ref, sums_scr, cnts_scr):
    i = pl.program_id(0)

    @pl.when(i == 0)
    def _():
        sums_scr[...] = jnp.zeros_like(sums_scr)
        cnts_scr[...] = jnp.zeros_like(cnts_scr)

    m = jnp.maximum(y_ref[...] + p0_ref[...] + p1_ref[...] + ba_ref[...], 0.0)
    h = jnp.maximum(
        jnp.dot(m, wb_ref[...], preferred_element_type=jnp.float32)
        + bb_ref[...], 0.0)
    b = batch_ref[0, 0, :]
    oht = (b[None, :] == lax.broadcasted_iota(jnp.int32, (_G, _BM), 0)
           ).astype(jnp.float32)
    sums_scr[...] += jnp.dot(oht, h, preferred_element_type=jnp.float32)
    cnts_scr[...] += jnp.sum(oht, axis=1)[:, None]

    @pl.when(i == _GRID - 1)
    def _():
        pooled = sums_scr[...] / jnp.maximum(cnts_scr[...], 1.0)
        z1 = jnp.maximum(
            jnp.dot(pooled, wc1_ref[...], preferred_element_type=jnp.float32)
            + bc1_ref[...], 0.0)
        z = jnp.dot(z1, wc2_ref[...],
                    preferred_element_type=jnp.float32) + bc2_ref[...]
        mx = jnp.max(z, axis=1, keepdims=True)
        e = jnp.exp(z - mx)
        o_ref[...] = z - mx - jnp.log(jnp.sum(e, axis=1, keepdims=True))


def _final(y, p0, p1, ba, wb, bb, batch_r, wc1, bc1, wc2, bc2):
    return pl.pallas_call(
        _final_body,
        grid=(_GRID,),
        in_specs=[
            pl.BlockSpec((_BM, _H), lambda i: (i, 0)),
            pl.BlockSpec((_BM, _H), lambda i: (i, 0)),
            pl.BlockSpec((_BM, _H), lambda i: (i, 0)),
            pl.BlockSpec((1, _H), lambda i: (0, 0)),
            pl.BlockSpec((_H, _H), lambda i: (0, 0)),
            pl.BlockSpec((1, _H), lambda i: (0, 0)),
            pl.BlockSpec((1, 1, _BM), lambda i: (i, 0, 0)),
            pl.BlockSpec((_H, _H), lambda i: (0, 0)),
            pl.BlockSpec((1, _H), lambda i: (0, 0)),
            pl.BlockSpec((_H, _O), lambda i: (0, 0)),
            pl.BlockSpec((1, _O), lambda i: (0, 0)),
        ],
        out_specs=pl.BlockSpec((_G, _O), lambda i: (0, 0)),
        out_shape=jax.ShapeDtypeStruct((_G, _O), jnp.float32),
        scratch_shapes=[
            pltpu.VMEM((_G, _H), jnp.float32),
            pltpu.VMEM((_G, 1), jnp.float32),
        ],
    )(y, p0, p1, ba.reshape(1, _H), wb, bb.reshape(1, _H), batch_r,
      wc1, bc1.reshape(1, _H), wc2, bc2.reshape(1, _O))


_sc_mesh = plsc.VectorSubcoreMesh(core_axis_name="c", subcore_axis_name="s")


@functools.partial(
    pl.kernel,
    mesh=_sc_mesh,
    compiler_params=pltpu.CompilerParams(use_tc_tiling_on_sc=False),
    out_type=(jax.ShapeDtypeStruct((_NPAD, _H), jnp.float32),
              jax.ShapeDtypeStruct((_NPAD, _H), jnp.float32)),
    scratch_types=[
        pltpu.VMEM((_KCH, _CHUNK), jnp.int32),
        pltpu.VMEM((_KCH, _CHUNK), jnp.int32),
        pltpu.VMEM((_NBUF, _CHUNK, _H), jnp.float32),
        pltpu.VMEM_SHARED((_NPAD, _H), jnp.float32),
        pltpu.VMEM_SHARED((_N, _H), jnp.float32),
        pltpu.SemaphoreType.DMA,
        pltpu.SemaphoreType.DMA,
    ],
)
def _sc_agg(y_hbm, zeros_hbm, src_hbm, dst_hbm, out0_hbm, out1_hbm,
            sidx, didx, rows, acc, y_sh, gsem, ssem):
    cid = lax.axis_index("c")
    sid = lax.axis_index("s")
    w = sid * 2 + cid
    r0 = sid * _RPT
    ry = _N // 16

    pltpu.sync_copy(zeros_hbm.at[pl.ds(r0, _RPT)], acc.at[pl.ds(r0, _RPT)])
    pltpu.sync_copy(y_hbm.at[pl.ds(sid * ry, ry)], y_sh.at[pl.ds(sid * ry, ry)])
    pltpu.sync_copy(src_hbm.at[pl.ds(w * _KCH, _KCH)], sidx)
    pltpu.sync_copy(dst_hbm.at[pl.ds(w * _KCH, _KCH)], didx)
    plsc.subcore_barrier()

    for b in range(_DEPTH):
        pltpu.async_copy(y_sh.at[sidx.at[b]], rows.at[b], gsem)

    def body(it, carry):
        base = it * _NBUF
        for b in range(_NBUF):
            j = base + b
            pltpu.make_async_copy(y_sh.at[sidx.at[j]], rows.at[b],
                                  gsem).wait()
            pltpu.async_copy(rows.at[b], acc.at[didx.at[j]], ssem, add=True)

            @pl.when(j >= _DEPTH)
            def _():
                pltpu.make_async_copy(rows.at[b], acc.at[didx.at[j]],
                                      ssem).wait()

            nj = j + _DEPTH
            nb = (b + _DEPTH) % _NBUF

            @pl.when(nj < _KCH)
            def _():
                pltpu.async_copy(y_sh.at[sidx.at[nj]], rows.at[nb], gsem)
        return carry

    lax.fori_loop(0, _KCH // _NBUF, body, 0)
    for _ in range(_DEPTH):
        pltpu.make_async_copy(rows.at[0], acc.at[didx.at[0]], ssem).wait()
    plsc.subcore_barrier()

    @pl.when(cid == 0)
    def _():
        pltpu.sync_copy(acc.at[pl.ds(r0, _RPT)], out0_hbm.at[pl.ds(r0, _RPT)])

    @pl.when(cid == 1)
    def _():
        pltpu.sync_copy(acc.at[pl.ds(r0, _RPT)], out1_hbm.at[pl.ds(r0, _RPT)])


def kernel(x, edge_index, batch, W0_0, b0_0, W0_1, b0_1, W1_0, b1_0,
           W1_1, b1_1, W2_0, b2_0, W2_1, b2_1, Wc1, bc1, Wc2, bc2):
    src = edge_index[0]
    dst = edge_index[1]
    pad = _EPAD - _E
    srcp = jnp.concatenate(
        [src, jnp.zeros((pad,), jnp.int32)]).reshape(_NW * _KCH, _CHUNK)
    trash = _N + (jnp.arange(pad, dtype=jnp.int32) % (_NPAD - _N))
    dstp = jnp.concatenate([dst, trash]).reshape(_NW * _KCH, _CHUNK)
    zeros = jnp.zeros((_NPAD, _H), jnp.float32)
    batch_r = batch.reshape(_GRID, 1, _BM)

    y0 = _project(x, W0_0)
    p0a, p0b = _sc_agg(y0, zeros, srcp, dstp)
    y1 = _mid(y0, p0a, p0b, b0_0, W0_1, b0_1, W1_0)
    p1a, p1b = _sc_agg(y1, zeros, srcp, dstp)
    y2 = _mid(y1, p1a, p1b, b1_0, W1_1, b1_1, W2_0)
    p2a, p2b = _sc_agg(y2, zeros, srcp, dstp)
    return _final(y2, p2a, p2b, b2_0, W2_1, b2_1, batch_r, Wc1, bc1, Wc2, bc2)


# speedup vs baseline: 21.7801x; 1.1145x over previous
import functools

import jax
import jax.numpy as jnp
from jax import lax
from jax.experimental import pallas as pl
from jax.experimental.pallas import tpu as pltpu
from jax.experimental.pallas import tpu_sc as plsc

_N, _E, _D, _H, _O, _G = 10000, 320000, 128, 32, 2, 64
_BM = 2000
_GRID = _N // _BM
_NW = 32
_CHUNK = 125
_KCH = 80
_EPT = _CHUNK * _KCH
_NPAD = 10112
_RPT = _NPAD // 16
_NBUF = 8
_DEPTH = 4


def _proj_body(x_ref, w_ref, o_ref):
    o_ref[...] = jnp.dot(x_ref[...], w_ref[...],
                         preferred_element_type=jnp.float32)


def _project(x, w0):
    return pl.pallas_call(
        _proj_body,
        grid=(_GRID,),
        in_specs=[
            pl.BlockSpec((_BM, _D), lambda i: (i, 0)),
            pl.BlockSpec((_D, _H), lambda i: (0, 0)),
        ],
        out_specs=pl.BlockSpec((_BM, _H), lambda i: (i, 0)),
        out_shape=jax.ShapeDtypeStruct((_N, _H), jnp.float32),
    )(x, w0)


def _mid_body(y_ref, p0_ref, p1_ref, ba_ref, wb_ref, bb_ref, wn_ref, o_ref):
    m = jnp.maximum(y_ref[...] + p0_ref[...] + p1_ref[...] + ba_ref[...], 0.0)
    h = jnp.maximum(
        jnp.dot(m, wb_ref[...], preferred_element_type=jnp.float32)
        + bb_ref[...], 0.0)
    o_ref[...] = jnp.dot(h, wn_ref[...], preferred_element_type=jnp.float32)


def _mid(y, p0, p1, ba, wb, bb, wn):
    return pl.pallas_call(
        _mid_body,
        grid=(_GRID,),
        in_specs=[
            pl.BlockSpec((_BM, _H), lambda i: (i, 0)),
            pl.BlockSpec((_BM, _H), lambda i: (i, 0)),
            pl.BlockSpec((_BM, _H), lambda i: (i, 0)),
            pl.BlockSpec((1, _H), lambda i: (0, 0)),
            pl.BlockSpec((_H, _H), lambda i: (0, 0)),
            pl.BlockSpec((1, _H), lambda i: (0, 0)),
            pl.BlockSpec((_H, _H), lambda i: (0, 0)),
        ],
        out_specs=pl.BlockSpec((_BM, _H), lambda i: (i, 0)),
        out_shape=jax.ShapeDtypeStruct((_N, _H), jnp.float32),
    )(y, p0, p1, ba.reshape(1, _H), wb, bb.reshape(1, _H), wn)


def _final_body(y_ref, p0_ref, p1_ref, ba_ref, wb_ref, bb_ref, batch_ref,
                wc1_ref, bc1_ref, wc2_ref, bc2_ref, o_ref, sums_scr, cnts_scr):
    i = pl.program_id(0)

    @pl.when(i == 0)
    def _():
        sums_scr[...] = jnp.zeros_like(sums_scr)
        cnts_scr[...] = jnp.zeros_like(cnts_scr)

    m = jnp.maximum(y_ref[...] + p0_ref[...] + p1_ref[...] + ba_ref[...], 0.0)
    h = jnp.maximum(
        jnp.dot(m, wb_ref[...], preferred_element_type=jnp.float32)
        + bb_ref[...], 0.0)
    b = batch_ref[0, 0, :]
    oht = (b[None, :] == lax.broadcasted_iota(jnp.int32, (_G, _BM), 0)
           ).astype(jnp.float32)
    sums_scr[...] += jnp.dot(oht, h, preferred_element_type=jnp.float32)
    cnts_scr[...] += jnp.sum(oht, axis=1)[:, None]

    @pl.when(i == _GRID - 1)
    def _():
        pooled = sums_scr[...] / jnp.maximum(cnts_scr[...], 1.0)
        z1 = jnp.maximum(
            jnp.dot(pooled, wc1_ref[...], preferred_element_type=jnp.float32)
            + bc1_ref[...], 0.0)
        z = jnp.dot(z1, wc2_ref[...],
                    preferred_element_type=jnp.float32) + bc2_ref[...]
        mx = jnp.max(z, axis=1, keepdims=True)
        e = jnp.exp(z - mx)
        o_ref[...] = z - mx - jnp.log(jnp.sum(e, axis=1, keepdims=True))


def _final(y, p0, p1, ba, wb, bb, batch_r, wc1, bc1, wc2, bc2):
    return pl.pallas_call(
        _final_body,
        grid=(_GRID,),
        in_specs=[
            pl.BlockSpec((_BM, _H), lambda i: (i, 0)),
            pl.BlockSpec((_BM, _H), lambda i: (i, 0)),
            pl.BlockSpec((_BM, _H), lambda i: (i, 0)),
            pl.BlockSpec((1, _H), lambda i: (0, 0)),
            pl.BlockSpec((_H, _H), lambda i: (0, 0)),
            pl.BlockSpec((1, _H), lambda i: (0, 0)),
            pl.BlockSpec((1, 1, _BM), lambda i: (i, 0, 0)),
            pl.BlockSpec((_H, _H), lambda i: (0, 0)),
            pl.BlockSpec((1, _H), lambda i: (0, 0)),
            pl.BlockSpec((_H, _O), lambda i: (0, 0)),
            pl.BlockSpec((1, _O), lambda i: (0, 0)),
        ],
        out_specs=pl.BlockSpec((_G, _O), lambda i: (0, 0)),
        out_shape=jax.ShapeDtypeStruct((_G, _O), jnp.float32),
        scratch_shapes=[
            pltpu.VMEM((_G, _H), jnp.float32),
            pltpu.VMEM((_G, 1), jnp.float32),
        ],
    )(y, p0, p1, ba.reshape(1, _H), wb, bb.reshape(1, _H), batch_r,
      wc1, bc1.reshape(1, _H), wc2, bc2.reshape(1, _O))


_sc_mesh = plsc.VectorSubcoreMesh(core_axis_name="c", subcore_axis_name="s")


@functools.partial(
    pl.kernel,
    mesh=_sc_mesh,
    compiler_params=pltpu.CompilerParams(use_tc_tiling_on_sc=False),
    out_type=(jax.ShapeDtypeStruct((_NPAD, _H), jnp.float32),
              jax.ShapeDtypeStruct((_NPAD, _H), jnp.float32)),
    scratch_types=[
        pltpu.VMEM((_KCH, _CHUNK), jnp.int32),
        pltpu.VMEM((_KCH, _CHUNK), jnp.int32),
        pltpu.VMEM((_NBUF, _CHUNK, _H), jnp.float32),
        pltpu.VMEM_SHARED((_NPAD, _H), jnp.float32),
        pltpu.VMEM_SHARED((_N, _H), jnp.float32),
        pltpu.SemaphoreType.DMA,
        pltpu.SemaphoreType.DMA,
    ],
)
def _sc_agg(y_hbm, zeros_hbm, ei_hbm, out0_hbm, out1_hbm,
            sidx, didx, rows, acc, y_sh, gsem, ssem):
    cid = lax.axis_index("c")
    sid = lax.axis_index("s")
    w = sid * 2 + cid
    r0 = sid * _RPT
    ry = _N // 16

    pltpu.sync_copy(zeros_hbm.at[pl.ds(r0, _RPT)], acc.at[pl.ds(r0, _RPT)])
    pltpu.sync_copy(y_hbm.at[pl.ds(sid * ry, ry)], y_sh.at[pl.ds(sid * ry, ry)])
    pltpu.sync_copy(ei_hbm.at[pl.ds(w * _KCH, _KCH)], sidx)
    pltpu.sync_copy(ei_hbm.at[pl.ds((_NW + w) * _KCH, _KCH)], didx)
    plsc.subcore_barrier()

    for b in range(_DEPTH):
        pltpu.async_copy(y_sh.at[sidx.at[b]], rows.at[b], gsem)

    def body(it, carry):
        base = it * _NBUF
        for b in range(_NBUF):
            j = base + b
            pltpu.make_async_copy(y_sh.at[sidx.at[j]], rows.at[b],
                                  gsem).wait()
            pltpu.async_copy(rows.at[b], acc.at[didx.at[j]], ssem, add=True)

            @pl.when(j >= _DEPTH)
            def _():
                pltpu.make_async_copy(rows.at[b], acc.at[didx.at[j]],
                                      ssem).wait()

            nj = j + _DEPTH
            nb = (b + _DEPTH) % _NBUF

            @pl.when(nj < _KCH)
            def _():
                pltpu.async_copy(y_sh.at[sidx.at[nj]], rows.at[nb], gsem)
        return carry

    lax.fori_loop(0, _KCH // _NBUF, body, 0)
    for _ in range(_DEPTH):
        pltpu.make_async_copy(rows.at[0], acc.at[didx.at[0]], ssem).wait()
    plsc.subcore_barrier()

    @pl.when(cid == 0)
    def _():
        pltpu.sync_copy(acc.at[pl.ds(r0, _RPT)], out0_hbm.at[pl.ds(r0, _RPT)])

    @pl.when(cid == 1)
    def _():
        pltpu.sync_copy(acc.at[pl.ds(r0, _RPT)], out1_hbm.at[pl.ds(r0, _RPT)])


def kernel(x, edge_index, batch, W0_0, b0_0, W0_1, b0_1, W1_0, b1_0,
           W1_1, b1_1, W2_0, b2_0, W2_1, b2_1, Wc1, bc1, Wc2, bc2):
    ei = edge_index.reshape(2 * _NW * _KCH, _CHUNK)
    zeros = jnp.zeros((_NPAD, _H), jnp.float32)
    batch_r = batch.reshape(_GRID, 1, _BM)

    y0 = _project(x, W0_0)
    p0a, p0b = _sc_agg(y0, zeros, ei)
    y1 = _mid(y0, p0a, p0b, b0_0, W0_1, b0_1, W1_0)
    p1a, p1b = _sc_agg(y1, zeros, ei)
    y2 = _mid(y1, p1a, p1b, b1_0, W1_1, b1_1, W2_0)
    p2a, p2b = _sc_agg(y2, zeros, ei)
    return _final(y2, p2a, p2b, b2_0, W2_1, b2_1, batch_r, Wc1, bc1, Wc2, bc2)
